# Initial kernel scaffold; baseline (speedup 1.0000x reference)
import jax
import jax.numpy as jnp
from jax.experimental import pallas as pl


def kernel(input, table):
    raise NotImplementedError("write your pallas kernel here")


# speedup vs baseline: 3.9572x; 3.9572x over previous
import functools

import jax
import jax.numpy as jnp
from jax import lax
from jax.experimental import pallas as pl
from jax.experimental.pallas import tpu as pltpu
from jax.experimental.pallas import tpu_sc as plsc

_NUM_CORES = 2
_NUM_SUBCORES = 16
_NUM_WORKERS = _NUM_CORES * _NUM_SUBCORES
_CHUNK = 512


def _gather_rows(table, idx_flat):
    num_indices = idx_flat.shape[0]
    embed_dim = table.shape[1]
    per_worker = num_indices // _NUM_WORKERS
    n_chunks = per_worker // _CHUNK

    mesh = plsc.VectorSubcoreMesh(core_axis_name="c", subcore_axis_name="s")

    @functools.partial(
        pl.kernel,
        mesh=mesh,
        out_type=jax.ShapeDtypeStruct((num_indices, embed_dim), table.dtype),
        scratch_types=[
            pltpu.VMEM((_CHUNK,), jnp.int32),
            pltpu.VMEM((_CHUNK, embed_dim), table.dtype),
            pltpu.SemaphoreType.DMA,
        ],
        compiler_params=pltpu.CompilerParams(use_tc_tiling_on_sc=False),
    )
    def k(table_hbm, idx_hbm, out_hbm, idx_v, rows_v, sem):
        wid = lax.axis_index("s") * _NUM_CORES + lax.axis_index("c")
        base = wid * per_worker

        @pl.loop(0, n_chunks)
        def _(c):
            off = base + c * _CHUNK
            pltpu.sync_copy(idx_hbm.at[pl.ds(off, _CHUNK)], idx_v)
            pltpu.async_copy(table_hbm.at[idx_v], rows_v, sem).wait()
            pltpu.sync_copy(rows_v, out_hbm.at[pl.ds(off, _CHUNK)])

    return k(table, idx_flat)


@jax.jit
def kernel(input, table):
    batch, hist = input.shape
    flat = input.reshape(batch * hist).astype(jnp.int32)
    out = _gather_rows(table, flat)
    return out.reshape(batch, hist, table.shape[1])


# speedup vs baseline: 4.2423x; 1.0720x over previous
import functools

import jax
import jax.numpy as jnp
from jax import lax
from jax.experimental import pallas as pl
from jax.experimental.pallas import tpu as pltpu
from jax.experimental.pallas import tpu_sc as plsc

_NUM_CORES = 2
_NUM_SUBCORES = 16
_NUM_WORKERS = _NUM_CORES * _NUM_SUBCORES
_CHUNK = 800


def _gather_rows(table, idx_flat):
    num_indices = idx_flat.shape[0]
    embed_dim = table.shape[1]
    per_worker = num_indices // _NUM_WORKERS
    n_chunks = per_worker // _CHUNK
    assert per_worker % _CHUNK == 0 and n_chunks % 2 == 0

    mesh = plsc.VectorSubcoreMesh(core_axis_name="c", subcore_axis_name="s")

    @functools.partial(
        pl.kernel,
        mesh=mesh,
        out_type=jax.ShapeDtypeStruct((num_indices, embed_dim), table.dtype),
        scratch_types=[
            pltpu.VMEM((_CHUNK,), jnp.int32),
            pltpu.VMEM((_CHUNK,), jnp.int32),
            pltpu.VMEM((_CHUNK, embed_dim), table.dtype),
            pltpu.VMEM((_CHUNK, embed_dim), table.dtype),
            pltpu.SemaphoreType.DMA,
            pltpu.SemaphoreType.DMA,
            pltpu.SemaphoreType.DMA,
            pltpu.SemaphoreType.DMA,
            pltpu.SemaphoreType.DMA,
            pltpu.SemaphoreType.DMA,
        ],
        compiler_params=pltpu.CompilerParams(use_tc_tiling_on_sc=False),
    )
    def k(table_hbm, idx_hbm, out_hbm, idx0, idx1, rows0, rows1,
          si0, si1, sg0, sg1, so0, so1):
        wid = lax.axis_index("s") * _NUM_CORES + lax.axis_index("c")
        base = wid * per_worker
        idx_v = (idx0, idx1)
        rows_v = (rows0, rows1)
        sem_i = (si0, si1)
        sem_g = (sg0, sg1)
        sem_o = (so0, so1)

        def idx_load(c, b):
            return pltpu.async_copy(
                idx_hbm.at[pl.ds(base + c * _CHUNK, _CHUNK)], idx_v[b],
                sem_i[b])

        def gather(c, b):
            del c
            return pltpu.async_copy(table_hbm.at[idx_v[b]], rows_v[b],
                                    sem_g[b])

        def store(c, b):
            return pltpu.async_copy(
                rows_v[b], out_hbm.at[pl.ds(base + c * _CHUNK, _CHUNK)],
                sem_o[b])

        idx_load(0, 0)
        idx_load(1, 1)
        pltpu.make_async_copy(idx_hbm.at[pl.ds(base, _CHUNK)], idx_v[0],
                              sem_i[0]).wait()
        gather(0, 0)

        @pl.loop(0, n_chunks, step=2)
        def _(g):
            for b in (0, 1):
                c = g + b
                pltpu.make_async_copy(table_hbm.at[idx_v[b]], rows_v[b],
                                      sem_g[b]).wait()

                @pl.when(c + 2 < n_chunks)
                def _():
                    idx_load(c + 2, b)

                store(c, b)

                @pl.when(c + 1 < n_chunks)
                def _():
                    pltpu.make_async_copy(
                        idx_hbm.at[pl.ds(base, _CHUNK)], idx_v[1 - b],
                        sem_i[1 - b]).wait()

                    @pl.when(c >= 1)
                    def _():
                        pltpu.make_async_copy(
                            rows_v[1 - b],
                            out_hbm.at[pl.ds(base, _CHUNK)],
                            sem_o[1 - b]).wait()

                    gather(c + 1, 1 - b)

        pltpu.make_async_copy(rows_v[0], out_hbm.at[pl.ds(base, _CHUNK)],
                              so0).wait()
        pltpu.make_async_copy(rows_v[1], out_hbm.at[pl.ds(base, _CHUNK)],
                              so1).wait()

    return k(table, idx_flat)


@jax.jit
def kernel(input, table):
    batch, hist = input.shape
    flat = input.reshape(batch * hist).astype(jnp.int32)
    out = _gather_rows(table, flat)
    return out.reshape(batch, hist, table.shape[1])
